# baseline (speedup 1.0000x reference)
import dataclasses
import functools

import jax
import jax.numpy as jnp
from jax import lax
from jax.experimental import pallas as pl
from jax.experimental.pallas import tpu as pltpu
from jax.experimental.pallas import tpu_sc as plsc

_M = 7
_NC = 2
_NS = 16
_NW = _NC * _NS
_LANES = 16


def _sc_conv(x1f, x2f, cf, total, chunk_groups):
    per_w = total // _NW
    cfloats = chunk_groups * _M
    n_chunks = per_w // cfloats
    n_vec = chunk_groups // _LANES

    mesh = plsc.VectorSubcoreMesh(core_axis_name="c", subcore_axis_name="s")
    cp = pltpu.CompilerParams()
    if "needs_layout_passes" in pltpu.CompilerParams.__dataclass_fields__:
        cp = dataclasses.replace(cp, needs_layout_passes=False)

    @functools.partial(
        pl.kernel,
        out_type=jax.ShapeDtypeStruct((total,), jnp.float32),
        mesh=mesh,
        compiler_params=cp,
        scratch_types=[
            pltpu.VMEM((cfloats,), jnp.float32),
            pltpu.VMEM((cfloats,), jnp.float32),
            pltpu.VMEM((cfloats,), jnp.float32),
            pltpu.VMEM((49 * _LANES,), jnp.float32),
        ],
    )
    def sc_k(x1_hbm, x2_hbm, c_hbm, out_hbm, x1_v, x2_v, out_v, c_v):
        wid = lax.axis_index("s") * _NC + lax.axis_index("c")
        base = wid * per_w
        pltpu.sync_copy(c_hbm, c_v)

        lane = lax.iota(jnp.int32, _LANES)
        lane7 = lane * _M
        csp = {}
        for m1 in range(_M):
            for m2 in range(_M - m1):
                csp[(m1, m2)] = plsc.load_gather(
                    c_v, [(m1 * _M + m2) * _LANES + lane])

        @pl.loop(0, n_chunks)
        def _chunk(j):
            off = base + j * cfloats
            pltpu.sync_copy(x1_hbm.at[pl.ds(off, cfloats)], x1_v)
            pltpu.sync_copy(x2_hbm.at[pl.ds(off, cfloats)], x2_v)

            @pl.loop(0, n_vec)
            def _vec(g):
                idx0 = g * (_LANES * _M) + lane7
                x1g = [plsc.load_gather(x1_v, [idx0 + m]) for m in range(_M)]
                x2g = [plsc.load_gather(x2_v, [idx0 + m]) for m in range(_M)]
                for k in range(_M):
                    acc = None
                    for m1 in range(k + 1):
                        t = x1g[m1] * x2g[k - m1] * csp[(m1, k - m1)]
                        acc = t if acc is None else acc + t
                    plsc.store_scatter(out_v, [idx0 + k], acc)

            pltpu.sync_copy(out_v, out_hbm.at[pl.ds(off, cfloats)])

    return sc_k(x1f, x2f, cf)


def kernel(X1, X2, clebsch):
    B, F, M = X1.shape
    total = B * F * M
    x1f = X1.reshape(total)
    x2f = X2.reshape(total)
    cf = jnp.repeat(clebsch.reshape(M * M), _LANES)
    out = _sc_conv(x1f, x2f, cf, total, chunk_groups=2048)
    return out.reshape(B, F, M)


# speedup vs baseline: 1.0029x; 1.0029x over previous
import dataclasses
import functools

import jax
import jax.numpy as jnp
from jax import lax
from jax.experimental import pallas as pl
from jax.experimental.pallas import tpu as pltpu
from jax.experimental.pallas import tpu_sc as plsc

_M = 7
_NC = 2
_NS = 16
_NW = _NC * _NS
_LANES = 16


def _sc_conv(x1f, x2f, cf, total, chunk_groups):
    per_w = total // _NW
    cfloats = chunk_groups * _M
    n_chunks = per_w // cfloats
    n_vec = chunk_groups // _LANES

    mesh = plsc.VectorSubcoreMesh(core_axis_name="c", subcore_axis_name="s")
    cp = pltpu.CompilerParams()
    if "needs_layout_passes" in pltpu.CompilerParams.__dataclass_fields__:
        cp = dataclasses.replace(cp, needs_layout_passes=False)

    @functools.partial(
        pl.kernel,
        out_type=jax.ShapeDtypeStruct((total,), jnp.float32),
        mesh=mesh,
        compiler_params=cp,
        scratch_types=[
            pltpu.VMEM((cfloats,), jnp.float32),
            pltpu.VMEM((cfloats,), jnp.float32),
            pltpu.VMEM((cfloats,), jnp.float32),
            pltpu.VMEM((49 * _LANES,), jnp.float32),
        ],
    )
    def sc_k(x1_hbm, x2_hbm, c_hbm, out_hbm, x1_v, x2_v, out_v, c_v):
        wid = lax.axis_index("s") * _NC + lax.axis_index("c")
        base = wid * per_w
        pltpu.sync_copy(c_hbm, c_v)

        lane = lax.iota(jnp.int32, _LANES)
        lane7 = lane * _M
        csp = {}
        for m1 in range(_M):
            for m2 in range(_M - m1):
                csp[(m1, m2)] = plsc.load_gather(
                    c_v, [(m1 * _M + m2) * _LANES + lane])

        @pl.loop(0, n_chunks)
        def _chunk(j):
            off = base + j * cfloats
            pltpu.sync_copy(x1_hbm.at[pl.ds(off, cfloats)], x1_v)
            pltpu.sync_copy(x2_hbm.at[pl.ds(off, cfloats)], x2_v)

            @plsc.parallel_loop(0, cfloats, _LANES * _M, unroll=4)
            def _vec(gbase):
                idx0 = gbase + lane7
                x1g = [plsc.load_gather(x1_v, [idx0 + m]) for m in range(_M)]
                x2g = [plsc.load_gather(x2_v, [idx0 + m]) for m in range(_M)]
                for k in range(_M):
                    acc = None
                    for m1 in range(k + 1):
                        t = x1g[m1] * x2g[k - m1] * csp[(m1, k - m1)]
                        acc = t if acc is None else acc + t
                    plsc.store_scatter(out_v, [idx0 + k], acc)

            pltpu.sync_copy(out_v, out_hbm.at[pl.ds(off, cfloats)])

    return sc_k(x1f, x2f, cf)


def kernel(X1, X2, clebsch):
    B, F, M = X1.shape
    total = B * F * M
    x1f = X1.reshape(total)
    x2f = X2.reshape(total)
    cf = jnp.repeat(clebsch.reshape(M * M), _LANES)
    out = _sc_conv(x1f, x2f, cf, total, chunk_groups=2048)
    return out.reshape(B, F, M)


# speedup vs baseline: 18.6201x; 18.5662x over previous
import dataclasses
import functools

import jax
import jax.numpy as jnp
from jax import lax
from jax.experimental import pallas as pl
from jax.experimental.pallas import tpu as pltpu
from jax.experimental.pallas import tpu_sc as plsc

_M = 7
_NC = 2
_NS = 16
_NW = _NC * _NS
_LANES = 16


def _sc_conv_planes(x1t, x2t, cf):
    _, F, B = x1t.shape
    W = B // _NW
    FT = F // 8

    mesh = plsc.VectorSubcoreMesh(core_axis_name="c", subcore_axis_name="s")
    cp = pltpu.CompilerParams(use_tc_tiling_on_sc=True)
    if "needs_layout_passes" in pltpu.CompilerParams.__dataclass_fields__:
        cp = dataclasses.replace(cp, needs_layout_passes=False)

    @functools.partial(
        pl.kernel,
        out_type=jax.ShapeDtypeStruct((_M, F, B), jnp.float32),
        mesh=mesh,
        compiler_params=cp,
        scratch_types=[
            pltpu.VMEM((_M, 8, W), jnp.float32),
            pltpu.VMEM((_M, 8, W), jnp.float32),
            pltpu.VMEM((_M, 8, W), jnp.float32),
            pltpu.VMEM((49 * _LANES,), jnp.float32),
        ],
    )
    def sc_k(x1_hbm, x2_hbm, c_hbm, out_hbm, x1_v, x2_v, out_v, c_v):
        wid = lax.axis_index("s") * _NC + lax.axis_index("c")
        b0 = wid * W
        pltpu.sync_copy(c_hbm, c_v)

        lane = lax.iota(jnp.int32, _LANES)
        csp = {}
        for m1 in range(_M):
            for m2 in range(_M - m1):
                csp[(m1, m2)] = plsc.load_gather(
                    c_v, [(m1 * _M + m2) * _LANES + lane])

        @pl.loop(0, FT)
        def _slab(ft):
            f0 = ft * 8
            pltpu.sync_copy(
                x1_hbm.at[:, pl.ds(f0, 8), pl.ds(b0, W)], x1_v)
            pltpu.sync_copy(
                x2_hbm.at[:, pl.ds(f0, 8), pl.ds(b0, W)], x2_v)

            for r in range(8):
                @plsc.parallel_loop(0, W, _LANES, unroll=2)
                def _vec(c0):
                    x1g = [x1_v[m, r, pl.ds(c0, _LANES)] for m in range(_M)]
                    x2g = [x2_v[m, r, pl.ds(c0, _LANES)] for m in range(_M)]
                    for k in range(_M):
                        acc = None
                        for m1 in range(k + 1):
                            t = x1g[m1] * x2g[k - m1] * csp[(m1, k - m1)]
                            acc = t if acc is None else acc + t
                        out_v[k, r, pl.ds(c0, _LANES)] = acc

            pltpu.sync_copy(
                out_v, out_hbm.at[:, pl.ds(f0, 8), pl.ds(b0, W)])

    return sc_k(x1t, x2t, cf)


def kernel(X1, X2, clebsch):
    B, F, M = X1.shape
    x1t = jnp.transpose(X1, (2, 1, 0))
    x2t = jnp.transpose(X2, (2, 1, 0))
    cf = jnp.repeat(clebsch.reshape(M * M), _LANES)
    out = _sc_conv_planes(x1t, x2t, cf)
    return jnp.transpose(out, (2, 1, 0))


# speedup vs baseline: 26.3758x; 1.4165x over previous
import dataclasses
import functools

import jax
import jax.numpy as jnp
from jax import lax
from jax.experimental import pallas as pl
from jax.experimental.pallas import tpu as pltpu
from jax.experimental.pallas import tpu_sc as plsc

_M = 7
_NC = 2
_NS = 16
_NW = _NC * _NS
_LANES = 16


def _sc_conv_planes(x1t, x2t, cf):
    _, F, B = x1t.shape
    W = 256
    FT = F // 8

    mesh = plsc.VectorSubcoreMesh(core_axis_name="c", subcore_axis_name="s")
    cp = pltpu.CompilerParams(use_tc_tiling_on_sc=True)
    if "needs_layout_passes" in pltpu.CompilerParams.__dataclass_fields__:
        cp = dataclasses.replace(cp, needs_layout_passes=False)

    @functools.partial(
        pl.kernel,
        out_type=jax.ShapeDtypeStruct((_M, F, B), jnp.float32),
        mesh=mesh,
        compiler_params=cp,
        scratch_types=[
            pltpu.VMEM((49 * _LANES,), jnp.float32),
        ],
    )
    def sc_k(x1_hbm, x2_hbm, c_hbm, out_hbm, c_v):
        pltpu.sync_copy(c_hbm, c_v)

        lane = lax.iota(jnp.int32, _LANES)
        csp = {}
        for m1 in range(_M):
            for m2 in range(_M - m1):
                csp[(m1, m2)] = plsc.load_gather(
                    c_v, [(m1 * _M + m2) * _LANES + lane])

        def body(x1_v, x2_v, out_v):
            for r in range(8):
                @plsc.parallel_loop(0, W, _LANES, unroll=2)
                def _vec(c0):
                    x1g = [x1_v[m, r, pl.ds(c0, _LANES)] for m in range(_M)]
                    x2g = [x2_v[m, r, pl.ds(c0, _LANES)] for m in range(_M)]
                    for k in range(_M):
                        acc = None
                        for m1 in range(k + 1):
                            t = x1g[m1] * x2g[k - m1] * csp[(m1, k - m1)]
                            acc = t if acc is None else acc + t
                        out_v[k, r, pl.ds(c0, _LANES)] = acc

        spec = pl.BlockSpec((_M, 8, W), lambda i, j: (0, i, j))
        pltpu.emit_pipeline(
            body,
            grid=(FT, B // W),
            in_specs=[spec, spec],
            out_specs=[spec],
            core_axis_name=("c", "s"),
            dimension_semantics=(pltpu.PARALLEL, pltpu.PARALLEL),
        )(x1_hbm, x2_hbm, out_hbm)

    return sc_k(x1t, x2t, cf)


def kernel(X1, X2, clebsch):
    B, F, M = X1.shape
    x1t = jnp.transpose(X1, (2, 1, 0))
    x2t = jnp.transpose(X2, (2, 1, 0))
    cf = jnp.repeat(clebsch.reshape(M * M), _LANES)
    out = _sc_conv_planes(x1t, x2t, cf)
    return jnp.transpose(out, (2, 1, 0))
